# baseline (speedup 1.0000x reference)
import functools
import jax
import jax.numpy as jnp
from jax import lax
from jax.experimental import pallas as pl
from jax.experimental.pallas import tpu as pltpu
from jax.experimental.pallas import tpu_sc as plsc

NBINS = 100
L = 16
HALF = 128
TC_ROWS = 96
SC_ROWS = 32
ROWS_PER_W = SC_ROWS // 32
ROWS_PER_BLOCK = 8


def _table_kernel(t_ref, mean_ref, std_ref, scale_ref, shift_ref):
    for r in range(128):
        tb = (t_ref[r] * NBINS).astype(jnp.int32)
        tb = jnp.where(tb == NBINS, NBINS - 1, tb)
        m = mean_ref[tb]
        s = std_ref[tb]
        scale_ref[r] = jnp.full((128,), 1.0 / s, dtype=jnp.float32)
        shift_ref[r] = jnp.full((128,), m, dtype=jnp.float32)


def _make_tables(t, data_mean, data_std):
    grid_spec = pltpu.PrefetchScalarGridSpec(
        num_scalar_prefetch=3,
        grid=(1,),
        in_specs=[],
        out_specs=[pl.BlockSpec((128, 128), lambda *_: (0, 0)),
                   pl.BlockSpec((128, 128), lambda *_: (0, 0))],
    )
    return pl.pallas_call(
        _table_kernel,
        grid_spec=grid_spec,
        out_shape=[jax.ShapeDtypeStruct((128, 128), jnp.float32),
                   jax.ShapeDtypeStruct((128, 128), jnp.float32)],
    )(t, data_mean, data_std)


def _norm_kernel(t_ref, mean_ref, std_ref, x_ref, o_ref):
    i = pl.program_id(0)
    for r in range(ROWS_PER_BLOCK):
        row = i * ROWS_PER_BLOCK + r
        tb = (t_ref[row] * NBINS).astype(jnp.int32)
        tb = jnp.where(tb == NBINS, NBINS - 1, tb)
        m = mean_ref[tb]
        s = std_ref[tb]
        o_ref[r] = (x_ref[r] - m) * (1.0 / s)


def _tc_part(x_t, t, data_mean, data_std):
    grid_spec = pltpu.PrefetchScalarGridSpec(
        num_scalar_prefetch=3,
        grid=(TC_ROWS // ROWS_PER_BLOCK,),
        in_specs=[pl.BlockSpec((ROWS_PER_BLOCK, 4, 256, 256),
                               lambda i, *_: (i, 0, 0, 0))],
        out_specs=pl.BlockSpec((ROWS_PER_BLOCK, 4, 256, 256),
                               lambda i, *_: (i, 0, 0, 0)),
    )
    return pl.pallas_call(
        _norm_kernel,
        grid_spec=grid_spec,
        out_shape=jax.ShapeDtypeStruct((TC_ROWS,) + x_t.shape[1:], x_t.dtype),
        compiler_params=pltpu.CompilerParams(
            dimension_semantics=("arbitrary",),
        ),
    )(t, data_mean, data_std, x_t)


def _sc_body(x_hbm, scale_hbm, shift_hbm, out_hbm,
             scale_v, shift_v, in_a):
    c = lax.axis_index("c")
    s = lax.axis_index("s")
    wid = s * 2 + c

    def row_loop(j, _):
        r = TC_ROWS + wid * ROWS_PER_W + j
        o = wid * ROWS_PER_W + j
        pltpu.sync_copy(scale_hbm.at[r], scale_v)
        pltpu.sync_copy(shift_hbm.at[r], shift_v)

        def chunk_loop(k, _):
            ch = k // 2
            h = (k % 2) * HALF
            pltpu.sync_copy(x_hbm.at[r, ch, pl.ds(h, HALF)], in_a)

            def col_loop(cg, _):
                col = cg * L
                rr = scale_v[pl.ds(0, L)]
                mm = shift_v[pl.ds(0, L)]
                for row in range(HALF):
                    v = in_a[row, pl.ds(col, L)]
                    in_a[row, pl.ds(col, L)] = (v - mm) * rr
                return 0
            lax.fori_loop(0, 256 // L, col_loop, 0)
            pltpu.sync_copy(in_a, out_hbm.at[o, ch, pl.ds(h, HALF)])
            return 0
        lax.fori_loop(0, 8, chunk_loop, 0)
        return 0
    lax.fori_loop(0, ROWS_PER_W, row_loop, 0)


def _sc_part(x_t, scale, shift):
    mesh = plsc.VectorSubcoreMesh(core_axis_name="c", subcore_axis_name="s")
    run = functools.partial(
        pl.kernel,
        mesh=mesh,
        out_type=jax.ShapeDtypeStruct((SC_ROWS,) + x_t.shape[1:], jnp.float32),
        scratch_types=[
            pltpu.VMEM((128,), jnp.float32),
            pltpu.VMEM((128,), jnp.float32),
            pltpu.VMEM((HALF, 256), jnp.float32),
        ],
    )(_sc_body)
    return run(x_t, scale, shift)


def kernel(x_t, t, data_mean, data_std):
    scale, shift = _make_tables(t, data_mean, data_std)
    sc_out = _sc_part(x_t, scale, shift)
    tc_out = _tc_part(x_t, t, data_mean, data_std)
    return jnp.concatenate([tc_out, sc_out], axis=0)


# speedup vs baseline: 2.2073x; 2.2073x over previous
import jax
import jax.numpy as jnp
from jax.experimental import pallas as pl
from jax.experimental.pallas import tpu as pltpu

NBINS = 100
ROWS_PER_BLOCK = 8


def _norm_kernel(t_ref, mean_ref, std_ref, x_ref, o_ref):
    i = pl.program_id(0)
    for r in range(ROWS_PER_BLOCK):
        row = i * ROWS_PER_BLOCK + r
        tb = (t_ref[row] * NBINS).astype(jnp.int32)
        tb = jnp.where(tb == NBINS, NBINS - 1, tb)
        m = mean_ref[tb]
        s = std_ref[tb]
        o_ref[r] = (x_ref[r] - m) * (1.0 / s)


def kernel(x_t, t, data_mean, data_std):
    B = x_t.shape[0]
    nb = B // ROWS_PER_BLOCK
    grid_spec = pltpu.PrefetchScalarGridSpec(
        num_scalar_prefetch=3,
        grid=(nb,),
        in_specs=[pl.BlockSpec((ROWS_PER_BLOCK, 4, 256, 256),
                               lambda i, *_: (i, 0, 0, 0))],
        out_specs=pl.BlockSpec((ROWS_PER_BLOCK, 4, 256, 256),
                               lambda i, *_: (i, 0, 0, 0)),
    )
    return pl.pallas_call(
        _norm_kernel,
        grid_spec=grid_spec,
        out_shape=jax.ShapeDtypeStruct(x_t.shape, x_t.dtype),
        compiler_params=pltpu.CompilerParams(
            dimension_semantics=("arbitrary",),
        ),
    )(t, data_mean, data_std, x_t)
